# Initial kernel scaffold; baseline (speedup 1.0000x reference)
import jax
import jax.numpy as jnp
from jax.experimental import pallas as pl


def kernel(x, adj, W, b):
    raise NotImplementedError("write your pallas kernel here")


# speedup vs baseline: 1.0724x; 1.0724x over previous
import functools

import jax
import jax.numpy as jnp
from jax.experimental import pallas as pl

N = 10000
F_IN = 512
N_CLS = 16
BM = 400


def _proj_body(x_ref, w_ref, o_ref):
    o_ref[...] = jnp.dot(x_ref[...], w_ref[...],
                         preferred_element_type=jnp.float32)


def _spmm_body(adj_ref, h_ref, o_ref):
    o_ref[...] = jnp.dot(adj_ref[...], h_ref[...],
                         preferred_element_type=jnp.float32)


def _spmm_final_body(adj_ref, h_ref, b_ref, o_ref):
    acc = jnp.dot(adj_ref[...], h_ref[...],
                  preferred_element_type=jnp.float32)
    acc = acc + b_ref[...]
    m = jnp.max(acc, axis=1, keepdims=True)
    z = acc - m
    lse = jnp.log(jnp.sum(jnp.exp(z), axis=1, keepdims=True))
    o_ref[...] = z - lse


@functools.partial(jax.jit, static_argnames=())
def kernel(x, adj, W, b):
    b2 = b.reshape(1, N_CLS)

    h0 = pl.pallas_call(
        _proj_body,
        out_shape=jax.ShapeDtypeStruct((N, N_CLS), jnp.float32),
        in_specs=[
            pl.BlockSpec((N, F_IN), lambda: (0, 0)),
            pl.BlockSpec((F_IN, N_CLS), lambda: (0, 0)),
        ],
        out_specs=pl.BlockSpec((N, N_CLS), lambda: (0, 0)),
    )(x, W)

    grid = (N // BM,)

    h1 = pl.pallas_call(
        _spmm_body,
        grid=grid,
        out_shape=jax.ShapeDtypeStruct((N, N_CLS), jnp.float32),
        in_specs=[
            pl.BlockSpec((BM, N), lambda i: (i, 0)),
            pl.BlockSpec((N, N_CLS), lambda i: (0, 0)),
        ],
        out_specs=pl.BlockSpec((BM, N_CLS), lambda i: (i, 0)),
    )(adj, h0)

    out = pl.pallas_call(
        _spmm_final_body,
        grid=grid,
        out_shape=jax.ShapeDtypeStruct((N, N_CLS), jnp.float32),
        in_specs=[
            pl.BlockSpec((BM, N), lambda i: (i, 0)),
            pl.BlockSpec((N, N_CLS), lambda i: (0, 0)),
            pl.BlockSpec((1, N_CLS), lambda i: (0, 0)),
        ],
        out_specs=pl.BlockSpec((BM, N_CLS), lambda i: (i, 0)),
    )(adj, h1, b2)

    return out


# speedup vs baseline: 1.0963x; 1.0223x over previous
import jax
import jax.numpy as jnp
from jax.experimental import pallas as pl
from jax.experimental.pallas import tpu as pltpu

N = 10000
F_IN = 512
N_CLS = 16
BM = 200
NBLK = N // BM


def _body(x_ref, w_ref, b_ref, adj_ref, o_ref, h_ref):
    step = pl.program_id(0)

    @pl.when(step == 0)
    def _():
        h_ref[pl.ds(0, N), :] = jnp.dot(
            x_ref[...], w_ref[...], preferred_element_type=jnp.float32)

    p = step // NBLK
    i = step - p * NBLK

    h = h_ref[pl.ds(p * N, N), :]
    acc = jnp.dot(adj_ref[...], h, preferred_element_type=jnp.float32)

    @pl.when(p == 0)
    def _():
        h_ref[pl.ds(N + i * BM, BM), :] = acc
        o_ref[...] = acc

    @pl.when(p == 1)
    def _():
        z = acc + b_ref[...]
        m = jnp.max(z, axis=1, keepdims=True)
        z = z - m
        lse = jnp.log(jnp.sum(jnp.exp(z), axis=1, keepdims=True))
        o_ref[...] = z - lse


def kernel(x, adj, W, b):
    b2 = b.reshape(1, N_CLS)
    return pl.pallas_call(
        _body,
        grid=(2 * NBLK,),
        out_shape=jax.ShapeDtypeStruct((N, N_CLS), jnp.float32),
        in_specs=[
            pl.BlockSpec((N, F_IN), lambda s: (0, 0)),
            pl.BlockSpec((F_IN, N_CLS), lambda s: (0, 0)),
            pl.BlockSpec((1, N_CLS), lambda s: (0, 0)),
            pl.BlockSpec((BM, N), lambda s: (s % NBLK, 0)),
        ],
        out_specs=pl.BlockSpec(
            (BM, N_CLS),
            lambda s: (jnp.where(s < NBLK, 0, s - NBLK), 0)),
        scratch_shapes=[pltpu.VMEM((2 * N, N_CLS), jnp.float32)],
    )(x, W, b2, adj)
